# Initial kernel scaffold; baseline (speedup 1.0000x reference)
import jax
import jax.numpy as jnp
from jax.experimental import pallas as pl


def kernel(points1, points2, features1, features2, W0, b0, g0, beta0, W1, b1, g1, beta1, W2, b2, g2, beta2):
    raise NotImplementedError("write your pallas kernel here")


# speedup vs baseline: 19.7497x; 19.7497x over previous
import functools

import jax
import jax.numpy as jnp
from jax.experimental import pallas as pl

_K = 16
_TILE = 512


def _dg(a, b, dims):
    return jax.lax.dot_general(a, b, (dims, ((), ())),
                               preferred_element_type=jnp.float32)


def _knn_l0_kernel(p2_ref, p1_ref, f1_ref, f2_ref, w0_ref, b0_ref,
                   y_ref, s_ref, ss_ref, *, n_ch):
    b = pl.program_id(0)
    i = pl.program_id(1)
    p2 = p2_ref[0]
    p1 = p1_ref[0]
    sq2 = jnp.sum(p2 * p2, axis=1, keepdims=True)
    sq1 = jnp.sum(p1 * p1, axis=0, keepdims=True)
    d2 = sq2 + sq1 - 2.0 * _dg(p2, p1, ((1,), (0,)))

    d2s = d2
    m = None
    for _ in range(_K):
        m = jnp.min(d2s, axis=1, keepdims=True)
        d2s = jnp.where(d2s <= m, jnp.float32(jnp.inf), d2s)
    mask = d2 <= m
    dist = jnp.sqrt(jnp.maximum(d2, 0.0))
    wr = jnp.where(mask, 1.0 / (dist + 1e-10), 0.0)
    w = wr / jnp.sum(wr, axis=1, keepdims=True)

    newf = _dg(w, f1_ref[0], ((1,), (0,)))
    f2 = f2_ref[0]
    w0 = w0_ref[...]
    c = newf.shape[1]
    y = (_dg(newf, w0[:, :c], ((1,), (1,)))
         + _dg(f2, w0[:, c:], ((1,), (1,)))
         + b0_ref[...])
    y_ref[0] = y

    @pl.when(jnp.logical_and(b == 0, i == 0))
    def _():
        s_ref[...] = jnp.zeros_like(s_ref)
        ss_ref[...] = jnp.zeros_like(ss_ref)

    s_ref[...] += jnp.sum(y, axis=0, keepdims=True)
    ss_ref[...] += jnp.sum(y * y, axis=0, keepdims=True)


def _bn_mlp_kernel(y_ref, s_ref, ss_ref, g_ref, be_ref, w_ref, b_ref,
                   o_ref, s2_ref, ss2_ref, *, ntot):
    i = pl.program_id(0)
    mean = s_ref[...] / ntot
    var = ss_ref[...] / ntot - mean * mean
    x = (y_ref[...] - mean) * jax.lax.rsqrt(var + 1e-3) * g_ref[...] + be_ref[...]
    h = jnp.maximum(x, 0.0)
    y2 = _dg(h, w_ref[...], ((1,), (1,))) + b_ref[...]

    o_ref[...] = y2

    @pl.when(i == 0)
    def _():
        s2_ref[...] = jnp.zeros_like(s2_ref)
        ss2_ref[...] = jnp.zeros_like(ss2_ref)

    s2_ref[...] += jnp.sum(y2, axis=0, keepdims=True)
    ss2_ref[...] += jnp.sum(y2 * y2, axis=0, keepdims=True)


def _bn_out_kernel(y_ref, s_ref, ss_ref, g_ref, be_ref, o_ref, *, ntot):
    mean = s_ref[...] / ntot
    var = ss_ref[...] / ntot - mean * mean
    x = (y_ref[...] - mean) * jax.lax.rsqrt(var + 1e-3) * g_ref[...] + be_ref[...]
    o_ref[...] = jnp.maximum(x, 0.0)


def kernel(points1, points2, features1, features2,
           W0, b0, g0, beta0, W1, b1, g1, beta1, W2, b2, g2, beta2):
    f32 = jnp.float32
    B, _, N1 = points1.shape
    N2 = points2.shape[2]
    C = features1.shape[1]
    OUT0 = W0.shape[0]
    OUT1 = W1.shape[0]
    OUT2 = W2.shape[0]
    T = min(_TILE, N2)
    M = B * N2

    pad1 = jnp.zeros((B, 8 - points1.shape[1], N1), f32)
    p1p = jnp.concatenate([points1, pad1], axis=1)
    p2p = jnp.concatenate(
        [points2, jnp.zeros((B, 8 - points2.shape[1], N2), f32)],
        axis=1).transpose(0, 2, 1)
    f1t = features1.transpose(0, 2, 1)
    f2t = features2.transpose(0, 2, 1)

    r2 = lambda v: v.reshape(1, -1)

    y0, s0, ss0 = pl.pallas_call(
        functools.partial(_knn_l0_kernel, n_ch=OUT0),
        grid=(B, N2 // T),
        in_specs=[
            pl.BlockSpec((1, T, 8), lambda b, i: (b, i, 0)),
            pl.BlockSpec((1, 8, N1), lambda b, i: (b, 0, 0)),
            pl.BlockSpec((1, N1, C), lambda b, i: (b, 0, 0)),
            pl.BlockSpec((1, T, C), lambda b, i: (b, i, 0)),
            pl.BlockSpec((OUT0, 2 * C), lambda b, i: (0, 0)),
            pl.BlockSpec((1, OUT0), lambda b, i: (0, 0)),
        ],
        out_specs=[
            pl.BlockSpec((1, T, OUT0), lambda b, i: (b, i, 0)),
            pl.BlockSpec((1, OUT0), lambda b, i: (0, 0)),
            pl.BlockSpec((1, OUT0), lambda b, i: (0, 0)),
        ],
        out_shape=[
            jax.ShapeDtypeStruct((B, N2, OUT0), f32),
            jax.ShapeDtypeStruct((1, OUT0), f32),
            jax.ShapeDtypeStruct((1, OUT0), f32),
        ],
    )(p2p, p1p, f1t, f2t, W0, r2(b0))

    def _layer(y, s, ss, g, be, W, bias, cin, cout):
        return pl.pallas_call(
            functools.partial(_bn_mlp_kernel, ntot=float(M)),
            grid=(M // T,),
            in_specs=[
                pl.BlockSpec((T, cin), lambda i: (i, 0)),
                pl.BlockSpec((1, cin), lambda i: (0, 0)),
                pl.BlockSpec((1, cin), lambda i: (0, 0)),
                pl.BlockSpec((1, cin), lambda i: (0, 0)),
                pl.BlockSpec((1, cin), lambda i: (0, 0)),
                pl.BlockSpec((cout, cin), lambda i: (0, 0)),
                pl.BlockSpec((1, cout), lambda i: (0, 0)),
            ],
            out_specs=[
                pl.BlockSpec((T, cout), lambda i: (i, 0)),
                pl.BlockSpec((1, cout), lambda i: (0, 0)),
                pl.BlockSpec((1, cout), lambda i: (0, 0)),
            ],
            out_shape=[
                jax.ShapeDtypeStruct((M, cout), f32),
                jax.ShapeDtypeStruct((1, cout), f32),
                jax.ShapeDtypeStruct((1, cout), f32),
            ],
        )(y, s, ss, r2(g), r2(be), W, r2(bias))

    y0 = y0.reshape(M, OUT0)
    y1, s1, ss1 = _layer(y0, s0, ss0, g0, beta0, W1, b1, OUT0, OUT1)
    y2, s2, ss2 = _layer(y1, s1, ss1, g1, beta1, W2, b2, OUT1, OUT2)

    (out,) = pl.pallas_call(
        functools.partial(_bn_out_kernel, ntot=float(M)),
        grid=(M // T,),
        in_specs=[
            pl.BlockSpec((T, OUT2), lambda i: (i, 0)),
            pl.BlockSpec((1, OUT2), lambda i: (0, 0)),
            pl.BlockSpec((1, OUT2), lambda i: (0, 0)),
            pl.BlockSpec((1, OUT2), lambda i: (0, 0)),
            pl.BlockSpec((1, OUT2), lambda i: (0, 0)),
        ],
        out_specs=[pl.BlockSpec((T, OUT2), lambda i: (i, 0))],
        out_shape=[jax.ShapeDtypeStruct((M, OUT2), f32)],
    )(y2, s2, ss2, r2(g2), r2(beta2))

    return out.reshape(B, N2, OUT2).transpose(0, 2, 1)


# speedup vs baseline: 21.5103x; 1.0891x over previous
import functools

import jax
import jax.numpy as jnp
from jax.experimental import pallas as pl

_K = 16
_TILE = 512


def _dg(a, b, dims):
    return jax.lax.dot_general(a, b, (dims, ((), ())),
                               preferred_element_type=jnp.float32)


def _knn_l0_kernel(p1_ref, p2_ref, f1_ref, f2_ref, w0_ref, b0_ref,
                   y_ref, s_ref, ss_ref):
    b = pl.program_id(0)
    i = pl.program_id(1)
    p1 = p1_ref[0]
    p2 = p2_ref[0]
    sq1 = jnp.sum(p1 * p1, axis=1, keepdims=True)
    sq2 = jnp.sum(p2 * p2, axis=0, keepdims=True)
    d2 = sq1 + sq2 - 2.0 * _dg(p1, p2, ((1,), (0,)))

    d2s = d2
    m = None
    for _ in range(_K):
        m = jnp.min(d2s, axis=0, keepdims=True)
        d2s = jnp.where(d2s <= m, jnp.float32(jnp.inf), d2s)
    mask = d2 <= m
    dist = jnp.sqrt(jnp.maximum(d2, 0.0))
    wr = jnp.where(mask, 1.0 / (dist + 1e-10), 0.0)
    w = wr / jnp.sum(wr, axis=0, keepdims=True)

    newf = _dg(f1_ref[0], w, ((1,), (0,)))
    f2 = f2_ref[0]
    w0 = w0_ref[...]
    c = newf.shape[0]
    y = (_dg(w0[:, :c], newf, ((1,), (0,)))
         + _dg(w0[:, c:], f2, ((1,), (0,)))
         + b0_ref[...])
    y_ref[0] = y

    @pl.when(jnp.logical_and(b == 0, i == 0))
    def _():
        s_ref[...] = jnp.zeros_like(s_ref)
        ss_ref[...] = jnp.zeros_like(ss_ref)

    s_ref[...] += jnp.sum(y, axis=1, keepdims=True)
    ss_ref[...] += jnp.sum(y * y, axis=1, keepdims=True)


def _bn_mlp_kernel(y_ref, s_ref, ss_ref, g_ref, be_ref, w_ref, b_ref,
                   o_ref, s2_ref, ss2_ref, *, ntot):
    b = pl.program_id(0)
    i = pl.program_id(1)
    mean = s_ref[...] / ntot
    var = ss_ref[...] / ntot - mean * mean
    x = (y_ref[0] - mean) * jax.lax.rsqrt(var + 1e-3) * g_ref[...] + be_ref[...]
    h = jnp.maximum(x, 0.0)
    y2 = _dg(w_ref[...], h, ((1,), (0,))) + b_ref[...]

    o_ref[0] = y2

    @pl.when(jnp.logical_and(b == 0, i == 0))
    def _():
        s2_ref[...] = jnp.zeros_like(s2_ref)
        ss2_ref[...] = jnp.zeros_like(ss2_ref)

    s2_ref[...] += jnp.sum(y2, axis=1, keepdims=True)
    ss2_ref[...] += jnp.sum(y2 * y2, axis=1, keepdims=True)


def _bn_out_kernel(y_ref, s_ref, ss_ref, g_ref, be_ref, o_ref, *, ntot):
    mean = s_ref[...] / ntot
    var = ss_ref[...] / ntot - mean * mean
    x = (y_ref[0] - mean) * jax.lax.rsqrt(var + 1e-3) * g_ref[...] + be_ref[...]
    o_ref[0] = jnp.maximum(x, 0.0)


def kernel(points1, points2, features1, features2,
           W0, b0, g0, beta0, W1, b1, g1, beta1, W2, b2, g2, beta2):
    f32 = jnp.float32
    B, _, N1 = points1.shape
    N2 = points2.shape[2]
    C = features1.shape[1]
    OUT0 = W0.shape[0]
    OUT1 = W1.shape[0]
    OUT2 = W2.shape[0]
    T = min(_TILE, N2)
    M = B * N2

    p1t = jnp.concatenate(
        [points1, jnp.zeros((B, 8 - points1.shape[1], N1), f32)],
        axis=1).transpose(0, 2, 1)
    p2p = jnp.concatenate(
        [points2, jnp.zeros((B, 8 - points2.shape[1], N2), f32)],
        axis=1)

    col = lambda v: v.reshape(-1, 1)

    y0, s0, ss0 = pl.pallas_call(
        _knn_l0_kernel,
        grid=(B, N2 // T),
        in_specs=[
            pl.BlockSpec((1, N1, 8), lambda b, i: (b, 0, 0)),
            pl.BlockSpec((1, 8, T), lambda b, i: (b, 0, i)),
            pl.BlockSpec((1, C, N1), lambda b, i: (b, 0, 0)),
            pl.BlockSpec((1, C, T), lambda b, i: (b, 0, i)),
            pl.BlockSpec((OUT0, 2 * C), lambda b, i: (0, 0)),
            pl.BlockSpec((OUT0, 1), lambda b, i: (0, 0)),
        ],
        out_specs=[
            pl.BlockSpec((1, OUT0, T), lambda b, i: (b, 0, i)),
            pl.BlockSpec((OUT0, 1), lambda b, i: (0, 0)),
            pl.BlockSpec((OUT0, 1), lambda b, i: (0, 0)),
        ],
        out_shape=[
            jax.ShapeDtypeStruct((B, OUT0, N2), f32),
            jax.ShapeDtypeStruct((OUT0, 1), f32),
            jax.ShapeDtypeStruct((OUT0, 1), f32),
        ],
    )(p1t, p2p, features1, features2, W0, col(b0))

    def _layer(y, s, ss, g, be, W, bias, cin, cout):
        return pl.pallas_call(
            functools.partial(_bn_mlp_kernel, ntot=float(M)),
            grid=(B, N2 // T),
            in_specs=[
                pl.BlockSpec((1, cin, T), lambda b, i: (b, 0, i)),
                pl.BlockSpec((cin, 1), lambda b, i: (0, 0)),
                pl.BlockSpec((cin, 1), lambda b, i: (0, 0)),
                pl.BlockSpec((cin, 1), lambda b, i: (0, 0)),
                pl.BlockSpec((cin, 1), lambda b, i: (0, 0)),
                pl.BlockSpec((cout, cin), lambda b, i: (0, 0)),
                pl.BlockSpec((cout, 1), lambda b, i: (0, 0)),
            ],
            out_specs=[
                pl.BlockSpec((1, cout, T), lambda b, i: (b, 0, i)),
                pl.BlockSpec((cout, 1), lambda b, i: (0, 0)),
                pl.BlockSpec((cout, 1), lambda b, i: (0, 0)),
            ],
            out_shape=[
                jax.ShapeDtypeStruct((B, cout, N2), f32),
                jax.ShapeDtypeStruct((cout, 1), f32),
                jax.ShapeDtypeStruct((cout, 1), f32),
            ],
        )(y, s, ss, col(g), col(be), W, col(bias))

    y1, s1, ss1 = _layer(y0, s0, ss0, g0, beta0, W1, b1, OUT0, OUT1)
    y2, s2, ss2 = _layer(y1, s1, ss1, g1, beta1, W2, b2, OUT1, OUT2)

    (out,) = pl.pallas_call(
        functools.partial(_bn_out_kernel, ntot=float(M)),
        grid=(B, N2 // T),
        in_specs=[
            pl.BlockSpec((1, OUT2, T), lambda b, i: (b, 0, i)),
            pl.BlockSpec((OUT2, 1), lambda b, i: (0, 0)),
            pl.BlockSpec((OUT2, 1), lambda b, i: (0, 0)),
            pl.BlockSpec((OUT2, 1), lambda b, i: (0, 0)),
            pl.BlockSpec((OUT2, 1), lambda b, i: (0, 0)),
        ],
        out_specs=[pl.BlockSpec((1, OUT2, T), lambda b, i: (b, 0, i))],
        out_shape=[jax.ShapeDtypeStruct((B, OUT2, N2), f32)],
    )(y2, s2, ss2, col(g2), col(beta2))

    return out
